# baseline (speedup 1.0000x reference)
import functools

import jax
import jax.numpy as jnp
from jax import lax
from jax.experimental import pallas as pl
from jax.experimental.pallas import tpu as pltpu
from jax.experimental.pallas import tpu_sc as plsc

_VEL_MAX_NORM = 1.0
_CTRL_MAX_NORM = 127.0

_B, _T, _D = 4096, 200, 128
_NFLAT = _B * _T * 2
_NC, _NS, _LANES = 2, 16, 16
_NW = _NC * _NS
_PER_W = _NFLAT // _NW
_CHUNK = 256
_NCHUNK = _PER_W // _CHUNK
_STREAM = 128
_SPC = _CHUNK // _STREAM


def _prescale_body(wv_ref, wc_ref, out_ref):
    wv = wv_ref[...]
    nv = jnp.max(jnp.abs(wv), axis=1, keepdims=True)
    sv = jnp.where(nv > _VEL_MAX_NORM,
                   _VEL_MAX_NORM / jnp.maximum(nv, 1e-12), 1.0)
    out_ref[0:_D, :] = wv * sv
    wc = wc_ref[...]
    nc = jnp.max(jnp.abs(wc), axis=1, keepdims=True)
    sc = jnp.where(nc > _CTRL_MAX_NORM,
                   _CTRL_MAX_NORM / jnp.maximum(nc, 1e-12), 1.0)
    out_ref[_D:2 * _D, :] = wc * sc


def _prescale(w_vel, w_ctrl):
    return pl.pallas_call(
        _prescale_body,
        out_shape=jax.ShapeDtypeStruct((2 * _D, _D), jnp.float32),
    )(w_vel, w_ctrl)


def _gather_body(table_hbm, idx_hbm, out_hbm, idx_v, rows_v, shared_tbl,
                 sem0, sem1, semw0, semw1):
    sems = (sem0, sem1)
    semws = (semw0, semw1)
    sid = lax.axis_index("s")
    wid = sid * _NC + lax.axis_index("c")
    base = wid * _PER_W

    @pl.when(sid == 0)
    def _():
        pltpu.sync_copy(table_hbm, shared_tbl)

    plsc.subcore_barrier()

    pltpu.sync_copy(idx_hbm.at[pl.ds(base, _PER_W)], idx_v)
    lane_bias = (lax.iota(jnp.int32, _LANES) % 2) * _D

    def _bias(i, _):
        off = i * _LANES
        idx_v[pl.ds(off, _LANES)] = idx_v[pl.ds(off, _LANES)] + lane_bias
        return 0

    lax.fori_loop(0, _PER_W // _LANES, _bias, 0)

    def _fire(c, b):
        for j in range(_SPC):
            pltpu.async_copy(
                shared_tbl.at[idx_v.at[pl.ds(c * _CHUNK + j * _STREAM, _STREAM)]],
                rows_v.at[b].at[pl.ds(j * _STREAM, _STREAM)],
                sems[b],
            )

    _fire(0, 0)

    def _wait_write(c, b):
        pltpu.make_async_copy(
            rows_v.at[b],
            out_hbm.at[pl.ds(base + c * _CHUNK, _CHUNK)],
            semws[b],
        ).wait()

    def _outer(gg, _):
        for b in range(2):
            c = gg * 2 + b

            @pl.when(c >= 1)
            def _():
                _wait_write(c - 1, 1 - b)

            @pl.when(c + 1 < _NCHUNK)
            def _():
                _fire(c + 1, 1 - b)

            pltpu.make_async_copy(
                out_hbm.at[pl.ds(base + c * _CHUNK, _CHUNK)],
                rows_v.at[b], sems[b],
            ).wait()
            pltpu.async_copy(rows_v.at[b],
                             out_hbm.at[pl.ds(base + c * _CHUNK, _CHUNK)],
                             semws[b])
        return 0

    lax.fori_loop(0, _NCHUNK // 2, _outer, 0)
    _wait_write(_NCHUNK - 1, 1)


def _gather(table, idx_flat):
    mesh = plsc.VectorSubcoreMesh(core_axis_name="c", subcore_axis_name="s")
    return pl.kernel(
        _gather_body,
        out_type=jax.ShapeDtypeStruct((_NFLAT, _D), jnp.float32),
        mesh=mesh,
        scratch_types=[
            pltpu.VMEM((_PER_W,), jnp.int32),
            pltpu.VMEM((2, _CHUNK, _D), jnp.float32),
            pltpu.VMEM_SHARED((2 * _D, _D), jnp.float32),
            pltpu.SemaphoreType.DMA,
            pltpu.SemaphoreType.DMA,
            pltpu.SemaphoreType.DMA,
            pltpu.SemaphoreType.DMA,
        ],
    )(table, idx_flat)


def kernel(x, W_vel, W_ctrl):
    table = _prescale(W_vel, W_ctrl)
    idx_flat = x.reshape(_NFLAT)
    out = _gather(table, idx_flat)
    return out.reshape(_B, _T, 2, _D)


# speedup vs baseline: 3.9829x; 3.9829x over previous
import jax
import jax.numpy as jnp
from jax import lax
from jax.experimental import pallas as pl
from jax.experimental.pallas import tpu as pltpu
from jax.experimental.pallas import tpu_sc as plsc

_VEL_MAX_NORM = 1.0
_CTRL_MAX_NORM = 127.0

_B, _T, _D = 4096, 200, 128
_NFLAT = _B * _T * 2
_NC, _NS, _LANES = 2, 16, 16
_NW = _NC * _NS
_NBLK = _B // _D
_PER_W = _NFLAT // _NW
_ROWS_PER_B = 2 * _T
_CHUNK = 256
_NCHUNK = _PER_W // _CHUNK
_STREAM = 128
_SPC = -(-_CHUNK // _STREAM)
_GPC = _CHUNK // _LANES
_STAGE_WAVE = 8
_NWAVE = _T // _STAGE_WAVE


def _prescale_body(wv_ref, wc_ref, out_ref):
    wv = wv_ref[...]
    nv = jnp.max(jnp.abs(wv), axis=1, keepdims=True)
    sv = jnp.where(nv > _VEL_MAX_NORM,
                   _VEL_MAX_NORM / jnp.maximum(nv, 1e-12), 1.0)
    out_ref[0:_D, :] = wv * sv
    wc = wc_ref[...]
    nc = jnp.max(jnp.abs(wc), axis=1, keepdims=True)
    sc = jnp.where(nc > _CTRL_MAX_NORM,
                   _CTRL_MAX_NORM / jnp.maximum(nc, 1e-12), 1.0)
    out_ref[_D:2 * _D, :] = wc * sc


def _prescale(w_vel, w_ctrl):
    return pl.pallas_call(
        _prescale_body,
        out_shape=jax.ShapeDtypeStruct((2 * _D, _D), jnp.float32),
    )(w_vel, w_ctrl)


def _gather_body(table_hbm, x1d_hbm, out_hbm, idx_v, cbuf, rows_v,
                 shared_tbl, sem0, sem1, semw0, semw1, semst, semtb):
    sems = (sem0, sem1)
    semws = (semw0, semw1)
    sid = lax.axis_index("s")
    wid = sid * _NC + lax.axis_index("c")
    base_row = wid * _PER_W

    @pl.when(sid == 0)
    def _():
        pltpu.async_copy(table_hbm, shared_tbl, semtb)

    def _fire_wave(wv):
        t0 = wv * _STAGE_WAVE
        for u in range(_STAGE_WAVE):
            t = t0 + u
            pltpu.async_copy(
                x1d_hbm.at[pl.ds((t * _NBLK + wid) * 2 * _D, 2 * _D)],
                idx_v.at[pl.ds(t * 2 * _D, 2 * _D)], semst)

    def _wait_wave(wv):
        t0 = wv * _STAGE_WAVE
        for u in range(_STAGE_WAVE):
            t = t0 + u
            pltpu.make_async_copy(
                x1d_hbm.at[pl.ds((t * _NBLK + wid) * 2 * _D, 2 * _D)],
                idx_v.at[pl.ds(t * 2 * _D, 2 * _D)], semst).wait()

    _fire_wave(0)

    def _stage_wave(wv, _):
        @pl.when(wv + 1 < _NWAVE)
        def _():
            _fire_wave(wv + 1)

        _wait_wave(wv)
        return 0

    lax.fori_loop(0, _NWAVE, _stage_wave, 0)

    @pl.when(sid == 0)
    def _():
        pltpu.make_async_copy(table_hbm, shared_tbl, semtb).wait()

    plsc.subcore_barrier()

    stride_vec = lax.iota(jnp.int32, _LANES) * _D
    bias_vec = (lax.iota(jnp.int32, _LANES) % 2) * _D

    def _prep(c, b):
        for g in range(_GPC):
            r0 = c * _CHUNK + g * _LANES
            b_local = r0 // _ROWS_PER_B
            k0 = r0 % _ROWS_PER_B
            pos = stride_vec + (k0 * _D + b_local)
            cbuf[b, pl.ds(g * _LANES, _LANES)] = (
                plsc.load_gather(idx_v, [pos]) + bias_vec)
        for j in range(_SPC):
            off = j * _STREAM
            n = min(_STREAM, _CHUNK - off)
            pltpu.async_copy(
                shared_tbl.at[cbuf.at[b].at[pl.ds(off, n)]],
                rows_v.at[b].at[pl.ds(off, n)],
                sems[b])

    _prep(0, 0)

    def _wait_write(c, b):
        pltpu.make_async_copy(
            rows_v.at[b],
            out_hbm.at[pl.ds(base_row + c * _CHUNK, _CHUNK)],
            semws[b],
        ).wait()

    def _outer(gg, _):
        for b in range(2):
            c = gg * 2 + b

            @pl.when(c >= 1)
            def _():
                _wait_write(c - 1, 1 - b)

            @pl.when(c + 1 < _NCHUNK)
            def _():
                _prep(c + 1, 1 - b)

            pltpu.make_async_copy(
                out_hbm.at[pl.ds(base_row + c * _CHUNK, _CHUNK)],
                rows_v.at[b], sems[b],
            ).wait()
            pltpu.async_copy(rows_v.at[b],
                             out_hbm.at[pl.ds(base_row + c * _CHUNK, _CHUNK)],
                             semws[b])
        return 0

    lax.fori_loop(0, _NCHUNK // 2, _outer, 0)
    _wait_write(_NCHUNK - 1, 1)


def _gather(table, x1d):
    mesh = plsc.VectorSubcoreMesh(core_axis_name="c", subcore_axis_name="s")
    return pl.kernel(
        _gather_body,
        out_type=jax.ShapeDtypeStruct((_NFLAT, _D), jnp.float32),
        mesh=mesh,
        compiler_params=pltpu.CompilerParams(needs_layout_passes=False),
        scratch_types=[
            pltpu.VMEM((_PER_W,), jnp.int32),
            pltpu.VMEM((2, _CHUNK), jnp.int32),
            pltpu.VMEM((2, _CHUNK, _D), jnp.float32),
            pltpu.VMEM_SHARED((2 * _D, _D), jnp.float32),
            pltpu.SemaphoreType.DMA,
            pltpu.SemaphoreType.DMA,
            pltpu.SemaphoreType.DMA,
            pltpu.SemaphoreType.DMA,
            pltpu.SemaphoreType.DMA,
            pltpu.SemaphoreType.DMA,
        ],
    )(table, x1d)


def kernel(x, W_vel, W_ctrl):
    table = _prescale(W_vel, W_ctrl)
    x1d = x.reshape(_NBLK, _D, _T, 2).transpose(2, 0, 3, 1).reshape(_NFLAT)
    out = _gather(table, x1d)
    return out.reshape(_B, _T, 2, _D)
